# baseline (speedup 1.0000x reference)
import functools

import jax
import jax.numpy as jnp
from jax import lax
from jax.experimental import pallas as pl
from jax.experimental.pallas import tpu as pltpu
from jax.experimental.pallas import tpu_sc as plsc

_NC = 2
_NS = 16
_NW = _NC * _NS


def kernel(image_feat, text_feat):
    n_rows, _ = image_feat.shape
    rows_per_w = n_rows // _NW
    mesh = plsc.VectorSubcoreMesh(
        core_axis_name="c", subcore_axis_name="s", num_cores=_NC
    )

    @functools.partial(
        pl.kernel,
        mesh=mesh,
        out_type=[
            jax.ShapeDtypeStruct(image_feat.shape, image_feat.dtype),
            jax.ShapeDtypeStruct(text_feat.shape, text_feat.dtype),
        ],
        scratch_types=[pltpu.SemaphoreType.DMA, pltpu.SemaphoreType.DMA],
    )
    def sc_copy(img_in, txt_in, img_out, txt_out, sem_i, sem_t):
        wid = lax.axis_index("s") * _NC + lax.axis_index("c")
        sl = pl.ds(wid * rows_per_w, rows_per_w)
        ci = pltpu.make_async_copy(img_in.at[sl], img_out.at[sl], sem_i)
        ct = pltpu.make_async_copy(txt_in.at[sl], txt_out.at[sl], sem_t)
        ci.start()
        ct.start()
        ci.wait()
        ct.wait()

    out = sc_copy(image_feat, text_feat)
    return (out[0], out[1])


# speedup vs baseline: 35.7354x; 35.7354x over previous
import functools

import jax
import jax.numpy as jnp
from jax import lax
from jax.experimental import pallas as pl
from jax.experimental.pallas import tpu as pltpu
from jax.experimental.pallas import tpu_sc as plsc

_NC = 2
_NS = 16
_NW = _NC * _NS
_CHUNK_ROWS = 32


def kernel(image_feat, text_feat):
    n_rows, n_cols = image_feat.shape
    rows_per_w = n_rows // _NW
    n_chunks = rows_per_w // _CHUNK_ROWS
    mesh = plsc.VectorSubcoreMesh(
        core_axis_name="c", subcore_axis_name="s", num_cores=_NC
    )

    @functools.partial(
        pl.kernel,
        mesh=mesh,
        out_type=[
            jax.ShapeDtypeStruct(image_feat.shape, image_feat.dtype),
            jax.ShapeDtypeStruct(text_feat.shape, text_feat.dtype),
        ],
        scratch_types=[
            pltpu.VMEM((2, _CHUNK_ROWS, n_cols), jnp.float32),
            pltpu.SemaphoreType.DMA((2,)),
            pltpu.SemaphoreType.DMA((2,)),
        ],
    )
    def sc_copy(img_in, txt_in, img_out, txt_out, bufs, ld_sem, st_sem):
        wid = lax.axis_index("s") * _NC + lax.axis_index("c")
        base = wid * rows_per_w

        lds, sts = [], []
        for src, dst in ((img_in, img_out), (txt_in, txt_out)):
            for c in range(n_chunks):
                k = len(lds)
                s = k % 2
                sl = pl.ds(base + c * _CHUNK_ROWS, _CHUNK_ROWS)
                lds.append(
                    pltpu.make_async_copy(src.at[sl], bufs.at[s], ld_sem.at[s])
                )
                sts.append(
                    pltpu.make_async_copy(bufs.at[s], dst.at[sl], st_sem.at[s])
                )

        total = len(lds)
        lds[0].start()
        for k in range(total):
            if k + 1 < total:
                if k >= 1:
                    sts[k - 1].wait()
                lds[k + 1].start()
            lds[k].wait()
            sts[k].start()
        sts[total - 2].wait()
        sts[total - 1].wait()

    out = sc_copy(image_feat, text_feat)
    return (out[0], out[1])


# speedup vs baseline: 38.8770x; 1.0879x over previous
import functools

import jax
import jax.numpy as jnp
from jax import lax
from jax.experimental import pallas as pl
from jax.experimental.pallas import tpu as pltpu
from jax.experimental.pallas import tpu_sc as plsc

_TC_BLOCK_ROWS = 2048

_NC = 2
_NS = 16
_NW = _NC * _NS
_SC_CHUNK_ROWS = 32


def _tc_body(src, dst):
    dst[...] = src[...]


def _tc_copy(x):
    n_rows, n_cols = x.shape
    spec = pl.BlockSpec((_TC_BLOCK_ROWS, n_cols), lambda i: (i, 0))
    return pl.pallas_call(
        _tc_body,
        grid=(n_rows // _TC_BLOCK_ROWS,),
        in_specs=[spec],
        out_specs=spec,
        out_shape=jax.ShapeDtypeStruct(x.shape, x.dtype),
    )(x)


def _sc_copy(x):
    n_rows, n_cols = x.shape
    rows_per_w = n_rows // _NW
    n_chunks = rows_per_w // _SC_CHUNK_ROWS
    mesh = plsc.VectorSubcoreMesh(
        core_axis_name="c", subcore_axis_name="s", num_cores=_NC
    )

    @functools.partial(
        pl.kernel,
        mesh=mesh,
        out_type=jax.ShapeDtypeStruct(x.shape, x.dtype),
        scratch_types=[
            pltpu.VMEM((2, _SC_CHUNK_ROWS, n_cols), jnp.float32),
            pltpu.SemaphoreType.DMA((2,)),
            pltpu.SemaphoreType.DMA((2,)),
        ],
    )
    def body(src, dst, bufs, ld_sem, st_sem):
        wid = lax.axis_index("s") * _NC + lax.axis_index("c")
        base = wid * rows_per_w

        lds, sts = [], []
        for c in range(n_chunks):
            s = c % 2
            sl = pl.ds(base + c * _SC_CHUNK_ROWS, _SC_CHUNK_ROWS)
            lds.append(pltpu.make_async_copy(src.at[sl], bufs.at[s], ld_sem.at[s]))
            sts.append(pltpu.make_async_copy(bufs.at[s], dst.at[sl], st_sem.at[s]))

        lds[0].start()
        for k in range(n_chunks):
            if k + 1 < n_chunks:
                if k >= 1:
                    sts[k - 1].wait()
                lds[k + 1].start()
            lds[k].wait()
            sts[k].start()
        sts[n_chunks - 2].wait()
        sts[n_chunks - 1].wait()

    return body(x)


def kernel(image_feat, text_feat):
    return (_tc_copy(image_feat), _sc_copy(text_feat))


# speedup vs baseline: 48.8431x; 1.2563x over previous
import jax
import jax.numpy as jnp
from jax.experimental import pallas as pl
from jax.experimental.pallas import tpu as pltpu

_BLOCK_ROWS = 1024


def _copy_body(img_in, txt_in, img_out, txt_out):
    img_out[...] = img_in[...]
    txt_out[...] = txt_in[...]


def kernel(image_feat, text_feat):
    n_rows, n_cols = image_feat.shape
    grid = (n_rows // _BLOCK_ROWS,)
    spec = pl.BlockSpec((_BLOCK_ROWS, n_cols), lambda i: (i, 0))
    out = pl.pallas_call(
        _copy_body,
        grid=grid,
        in_specs=[spec, spec],
        out_specs=[spec, spec],
        out_shape=[
            jax.ShapeDtypeStruct(image_feat.shape, image_feat.dtype),
            jax.ShapeDtypeStruct(text_feat.shape, text_feat.dtype),
        ],
        compiler_params=pltpu.CompilerParams(
            dimension_semantics=("parallel",),
        ),
    )(image_feat, text_feat)
    return (out[0], out[1])


# speedup vs baseline: 49.4649x; 1.0127x over previous
import jax
import jax.numpy as jnp
from jax.experimental import pallas as pl
from jax.experimental.pallas import tpu as pltpu

_CHUNK_ROWS = 1024
_K = 3


def _copy_body(img_in, txt_in, img_out, txt_out, buf_i, buf_t, ld_i, ld_t, st_i, st_t):
    n_chunks = img_in.shape[0] // _CHUNK_ROWS

    streams = []
    for src, dst, buf, ld_sem, st_sem in (
        (img_in, img_out, buf_i, ld_i, st_i),
        (txt_in, txt_out, buf_t, ld_t, st_t),
    ):
        lds, sts = [], []
        for c in range(n_chunks):
            s = c % _K
            sl = pl.ds(c * _CHUNK_ROWS, _CHUNK_ROWS)
            lds.append(pltpu.make_async_copy(src.at[sl], buf.at[s], ld_sem.at[s]))
            sts.append(pltpu.make_async_copy(buf.at[s], dst.at[sl], st_sem.at[s]))
        streams.append((lds, sts))

    for k in range(_K):
        for lds, _ in streams:
            lds[k].start()
    for k in range(n_chunks):
        for lds, sts in streams:
            lds[k].wait()
            sts[k].start()
        if k + _K < n_chunks:
            for lds, sts in streams:
                sts[k].wait()
                lds[k + _K].start()
    for k in range(n_chunks - _K, n_chunks):
        for _, sts in streams:
            sts[k].wait()


def kernel(image_feat, text_feat):
    n_cols = image_feat.shape[1]
    out = pl.pallas_call(
        _copy_body,
        in_specs=[
            pl.BlockSpec(memory_space=pl.MemorySpace.ANY),
            pl.BlockSpec(memory_space=pl.MemorySpace.ANY),
        ],
        out_specs=[
            pl.BlockSpec(memory_space=pl.MemorySpace.ANY),
            pl.BlockSpec(memory_space=pl.MemorySpace.ANY),
        ],
        out_shape=[
            jax.ShapeDtypeStruct(image_feat.shape, image_feat.dtype),
            jax.ShapeDtypeStruct(text_feat.shape, text_feat.dtype),
        ],
        scratch_shapes=[
            pltpu.VMEM((_K, _CHUNK_ROWS, n_cols), jnp.float32),
            pltpu.VMEM((_K, _CHUNK_ROWS, n_cols), jnp.float32),
            pltpu.SemaphoreType.DMA((_K,)),
            pltpu.SemaphoreType.DMA((_K,)),
            pltpu.SemaphoreType.DMA((_K,)),
            pltpu.SemaphoreType.DMA((_K,)),
        ],
    )(image_feat, text_feat)
    return (out[0], out[1])
